# baseline (speedup 1.0000x reference)
import functools

import jax
import jax.numpy as jnp
from jax import lax
from jax.experimental import pallas as pl
from jax.experimental.pallas import tpu as pltpu
from jax.experimental.pallas import tpu_sc as plsc

B = 16
T = 512
D = 512
H3 = 3 * D
A = 256

NCHUNK = 4
CT = T // NCHUNK

_NC = 2
_NS = 16
_NW = _NC * _NS

_PREC = lax.Precision.DEFAULT


def _sc_gather(table, idx):
    n = idx.shape[0]
    d = table.shape[1]
    per_w = n // _NW
    chunk = min(per_w, 64)
    mesh = plsc.VectorSubcoreMesh(core_axis_name="c", subcore_axis_name="s")

    @functools.partial(
        pl.kernel,
        mesh=mesh,
        out_type=jax.ShapeDtypeStruct((n, d), table.dtype),
        scratch_types=[
            pltpu.VMEM((per_w,), jnp.int32),
            pltpu.VMEM((chunk, d), table.dtype),
            pltpu.VMEM((chunk, d), table.dtype),
            pltpu.SemaphoreType.DMA,
            pltpu.SemaphoreType.DMA,
        ],
    )
    def k(table_hbm, idx_hbm, out_hbm, idx_v, rows0, rows1, sem0, sem1):
        wid = lax.axis_index("s") * _NC + lax.axis_index("c")
        base = wid * per_w
        pltpu.sync_copy(idx_hbm.at[pl.ds(base, per_w)], idx_v)
        bufs = (rows0, rows1)
        sems = (sem0, sem1)
        nchunks = per_w // chunk
        pltpu.async_copy(table_hbm.at[idx_v.at[pl.ds(0, chunk)]], bufs[0], sems[0])
        for c in range(nchunks):
            if c + 1 < nchunks:
                pltpu.async_copy(
                    table_hbm.at[idx_v.at[pl.ds((c + 1) * chunk, chunk)]],
                    bufs[(c + 1) % 2], sems[(c + 1) % 2])
            pltpu.make_async_copy(
                table_hbm.at[idx_v.at[pl.ds(c * chunk, chunk)]],
                bufs[c % 2], sems[c % 2]).wait()
            pltpu.sync_copy(bufs[c % 2], out_hbm.at[pl.ds(base + c * chunk, chunk)])

    return k(table, idx)


def _sig(x):
    return 0.5 * jnp.tanh(0.5 * x) + 0.5


def _chunk_fwd(cidx, e_sm, emb, w_ih_t, b_ih, w_hh_t, b_hh, h0, out0):
    st = 64
    mm_rows = st * B
    ngrid = CT // st

    def body(e_ref, emb_ref, wih_ref, bih_ref, whh_ref, bhh_ref,
             h0_ref, out0_ref, h1_ref, out1_ref, gi_ref, hall_ref, h_ref):
        g = pl.program_id(0)

        @pl.when(g == 0)
        def _():
            h_ref[...] = h0_ref[...]
            out1_ref[...] = out0_ref[...]

        gi = lax.dot_general(
            emb_ref[...].astype(jnp.bfloat16),
            wih_ref[...], (((1,), (0,)), ((), ())),
            preferred_element_type=jnp.float32, precision=_PREC,
        ) + bih_ref[...]
        gi_ref[...] = gi.astype(jnp.bfloat16).reshape(st, B, H3)

        def step(i, h):
            gg = gi_ref[i].astype(jnp.float32)
            hb = h.astype(jnp.bfloat16)
            gh_rz = lax.dot_general(
                hb, whh_ref[:, :2 * D], (((1,), (0,)), ((), ())),
                preferred_element_type=jnp.float32, precision=_PREC,
            ) + bhh_ref[:, :2 * D]
            gh_n = lax.dot_general(
                hb, whh_ref[:, 2 * D:], (((1,), (0,)), ((), ())),
                preferred_element_type=jnp.float32, precision=_PREC,
            ) + bhh_ref[:, 2 * D:]
            r = _sig(gg[:, :D] + gh_rz[:, :D])
            z = _sig(gg[:, D:2 * D] + gh_rz[:, D:2 * D])
            nn = jnp.tanh(gg[:, 2 * D:] + r * gh_n)
            h = (1.0 - z) * nn + z * h
            hall_ref[i, :, :] = h
            return h

        h = lax.fori_loop(0, st, step, h_ref[...], unroll=4)
        h_ref[...] = h
        h1_ref[...] = h
        base = cidx * CT + g * st
        for b in range(B):
            eb = e_ref[0, b]

            @pl.when(jnp.logical_and(eb >= base, eb < base + st))
            def _():
                out1_ref[b, :] = hall_ref[eb - base, b, :]

    return pl.pallas_call(
        body,
        grid=(ngrid,),
        in_specs=[
            pl.BlockSpec(memory_space=pltpu.SMEM),
            pl.BlockSpec((mm_rows, D), lambda g: (g, 0)),
            pl.BlockSpec((D, H3), lambda g: (0, 0)),
            pl.BlockSpec((1, H3), lambda g: (0, 0)),
            pl.BlockSpec((D, H3), lambda g: (0, 0)),
            pl.BlockSpec((1, H3), lambda g: (0, 0)),
            pl.BlockSpec((B, D), lambda g: (0, 0)),
            pl.BlockSpec((B, D), lambda g: (0, 0)),
        ],
        out_specs=(
            pl.BlockSpec((B, D), lambda g: (0, 0)),
            pl.BlockSpec((B, D), lambda g: (0, 0)),
        ),
        out_shape=(
            jax.ShapeDtypeStruct((B, D), jnp.float32),
            jax.ShapeDtypeStruct((B, D), jnp.float32),
        ),
        scratch_shapes=[
            pltpu.VMEM((st, B, H3), jnp.bfloat16),
            pltpu.VMEM((st, B, D), jnp.float32),
            pltpu.VMEM((B, D), jnp.float32),
        ],
    )(e_sm, emb, w_ih_t, b_ih.reshape(1, H3), w_hh_t, b_hh.reshape(1, H3),
      h0, out0)


def _first_zero(utterance):
    def body(u_ref, e_ref):
        u = u_ref[...]
        iota = lax.broadcasted_iota(jnp.int32, (B, T), 1)
        cand = jnp.where(u == 0, iota, T - 1)
        e_ref[...] = jnp.min(cand, axis=1)[None, :]

    return pl.pallas_call(
        body, out_shape=jax.ShapeDtypeStruct((1, B), jnp.int32)
    )(utterance)


def _finalize(out_state, w_out_t, b_out, gumbel):
    def body(h_ref, w_ref, b_ref, g_ref, probs_ref, act_ref):
        logits = lax.dot_general(
            h_ref[...].astype(jnp.bfloat16), w_ref[...],
            (((1,), (0,)), ((), ())),
            preferred_element_type=jnp.float32, precision=_PREC,
        ) + b_ref[...]
        mx = jnp.max(logits, axis=1, keepdims=True)
        ex = jnp.exp(logits - mx)
        probs = ex / jnp.sum(ex, axis=1, keepdims=True)
        probs_ref[...] = probs
        y = jnp.log(probs + 1e-20) + g_ref[...]
        ymax = jnp.max(y, axis=1, keepdims=True)
        aio = lax.broadcasted_iota(jnp.int32, (B, A), 1)
        act_ref[...] = jnp.min(jnp.where(y == ymax, aio, A), axis=1)[None, :]

    return pl.pallas_call(
        body,
        out_shape=(
            jax.ShapeDtypeStruct((B, A), jnp.float32),
            jax.ShapeDtypeStruct((1, B), jnp.int32),
        ),
    )(out_state, w_out_t, b_out.reshape(1, A), gumbel)


def kernel(utterance, global_idxes, emb_table, W_ih, W_hh, b_ih, b_hh, W_out, b_out):
    del global_idxes
    tokens_tm = utterance.T.reshape(-1)
    wih = W_ih.T.astype(jnp.bfloat16)
    whh = W_hh.T.astype(jnp.bfloat16)
    e = _first_zero(utterance)
    h = jnp.zeros((B, D), jnp.float32)
    out = jnp.zeros((B, D), jnp.float32)
    for c in range(NCHUNK):
        emb_c = _sc_gather(emb_table, tokens_tm[c * CT * B:(c + 1) * CT * B])
        h, out = _chunk_fwd(c, e, emb_c, wih, b_ih, whh, b_hh, h, out)
    gumbel = jax.random.gumbel(jax.random.key(42), (B, A), jnp.float32)
    probs, act = _finalize(out, W_out.T.astype(jnp.bfloat16), b_out, gumbel)
    return probs, act.reshape(B)


# speedup vs baseline: 1.0066x; 1.0066x over previous
import functools

import jax
import jax.numpy as jnp
from jax import lax
from jax.experimental import pallas as pl
from jax.experimental.pallas import tpu as pltpu
from jax.experimental.pallas import tpu_sc as plsc

B = 16
T = 512
D = 512
H3 = 3 * D
A = 256

NCHUNK = 4
CT = T // NCHUNK

_NC = 2
_NS = 16
_NW = _NC * _NS

_PREC = lax.Precision.DEFAULT


def _sc_gather(table, idx):
    n = idx.shape[0]
    d = table.shape[1]
    per_w = n // _NW
    chunk = max(c for c in range(1, min(per_w, 64) + 1) if per_w % c == 0)
    mesh = plsc.VectorSubcoreMesh(core_axis_name="c", subcore_axis_name="s")

    @functools.partial(
        pl.kernel,
        mesh=mesh,
        out_type=jax.ShapeDtypeStruct((n, d), table.dtype),
        scratch_types=[
            pltpu.VMEM((per_w,), jnp.int32),
            pltpu.VMEM((chunk, d), table.dtype),
            pltpu.VMEM((chunk, d), table.dtype),
            pltpu.SemaphoreType.DMA,
            pltpu.SemaphoreType.DMA,
        ],
    )
    def k(table_hbm, idx_hbm, out_hbm, idx_v, rows0, rows1, sem0, sem1):
        wid = lax.axis_index("s") * _NC + lax.axis_index("c")
        base = wid * per_w
        pltpu.sync_copy(idx_hbm.at[pl.ds(base, per_w)], idx_v)
        bufs = (rows0, rows1)
        sems = (sem0, sem1)
        nchunks = per_w // chunk
        pltpu.async_copy(table_hbm.at[idx_v.at[pl.ds(0, chunk)]], bufs[0], sems[0])
        for c in range(nchunks):
            if c + 1 < nchunks:
                pltpu.async_copy(
                    table_hbm.at[idx_v.at[pl.ds((c + 1) * chunk, chunk)]],
                    bufs[(c + 1) % 2], sems[(c + 1) % 2])
            pltpu.make_async_copy(
                table_hbm.at[idx_v.at[pl.ds(c * chunk, chunk)]],
                bufs[c % 2], sems[c % 2]).wait()
            pltpu.sync_copy(bufs[c % 2], out_hbm.at[pl.ds(base + c * chunk, chunk)])

    return k(table, idx)


def _sig(x):
    return 0.5 * jnp.tanh(0.5 * x) + 0.5


def _chunk_fwd(base0, ct, e_sm, emb, w_ih_t, b_ih, w_hh_t, b_hh, h0, out0):
    st = 64
    mm_rows = st * B
    ngrid = ct // st

    def body(e_ref, emb_ref, wih_ref, bih_ref, whh_ref, bhh_ref,
             h0_ref, out0_ref, h1_ref, out1_ref, gi_ref, hall_ref, h_ref):
        g = pl.program_id(0)

        @pl.when(g == 0)
        def _():
            h_ref[...] = h0_ref[...]
            out1_ref[...] = out0_ref[...]

        gi = lax.dot_general(
            emb_ref[...].astype(jnp.bfloat16),
            wih_ref[...], (((1,), (0,)), ((), ())),
            preferred_element_type=jnp.float32, precision=_PREC,
        ) + bih_ref[...]
        gi_ref[...] = gi.astype(jnp.bfloat16).reshape(st, B, H3)

        def step(i, h):
            gg = gi_ref[i].astype(jnp.float32)
            hb = h.astype(jnp.bfloat16)
            gh_rz = lax.dot_general(
                hb, whh_ref[:, :2 * D], (((1,), (0,)), ((), ())),
                preferred_element_type=jnp.float32, precision=_PREC,
            ) + bhh_ref[:, :2 * D]
            gh_n = lax.dot_general(
                hb, whh_ref[:, 2 * D:], (((1,), (0,)), ((), ())),
                preferred_element_type=jnp.float32, precision=_PREC,
            ) + bhh_ref[:, 2 * D:]
            r = _sig(gg[:, :D] + gh_rz[:, :D])
            z = _sig(gg[:, D:2 * D] + gh_rz[:, D:2 * D])
            nn = jnp.tanh(gg[:, 2 * D:] + r * gh_n)
            h = (1.0 - z) * nn + z * h
            hall_ref[i, :, :] = h
            return h

        h = lax.fori_loop(0, st, step, h_ref[...], unroll=4)
        h_ref[...] = h
        h1_ref[...] = h
        base = base0 + g * st
        for b in range(B):
            eb = e_ref[0, b]

            @pl.when(jnp.logical_and(eb >= base, eb < base + st))
            def _():
                out1_ref[b, :] = hall_ref[eb - base, b, :]

    return pl.pallas_call(
        body,
        grid=(ngrid,),
        in_specs=[
            pl.BlockSpec(memory_space=pltpu.SMEM),
            pl.BlockSpec((mm_rows, D), lambda g: (g, 0)),
            pl.BlockSpec((D, H3), lambda g: (0, 0)),
            pl.BlockSpec((1, H3), lambda g: (0, 0)),
            pl.BlockSpec((D, H3), lambda g: (0, 0)),
            pl.BlockSpec((1, H3), lambda g: (0, 0)),
            pl.BlockSpec((B, D), lambda g: (0, 0)),
            pl.BlockSpec((B, D), lambda g: (0, 0)),
        ],
        out_specs=(
            pl.BlockSpec((B, D), lambda g: (0, 0)),
            pl.BlockSpec((B, D), lambda g: (0, 0)),
        ),
        out_shape=(
            jax.ShapeDtypeStruct((B, D), jnp.float32),
            jax.ShapeDtypeStruct((B, D), jnp.float32),
        ),
        scratch_shapes=[
            pltpu.VMEM((st, B, H3), jnp.bfloat16),
            pltpu.VMEM((st, B, D), jnp.float32),
            pltpu.VMEM((B, D), jnp.float32),
        ],
    )(e_sm, emb, w_ih_t, b_ih.reshape(1, H3), w_hh_t, b_hh.reshape(1, H3),
      h0, out0)


def _first_zero(utterance):
    def body(u_ref, e_ref):
        u = u_ref[...]
        iota = lax.broadcasted_iota(jnp.int32, (B, T), 1)
        cand = jnp.where(u == 0, iota, T - 1)
        e_ref[...] = jnp.min(cand, axis=1)[None, :]

    return pl.pallas_call(
        body, out_shape=jax.ShapeDtypeStruct((1, B), jnp.int32)
    )(utterance)


def _finalize(out_state, w_out_t, b_out, gumbel):
    def body(h_ref, w_ref, b_ref, g_ref, probs_ref, act_ref):
        logits = lax.dot_general(
            h_ref[...].astype(jnp.bfloat16), w_ref[...],
            (((1,), (0,)), ((), ())),
            preferred_element_type=jnp.float32, precision=_PREC,
        ) + b_ref[...]
        mx = jnp.max(logits, axis=1, keepdims=True)
        ex = jnp.exp(logits - mx)
        probs = ex / jnp.sum(ex, axis=1, keepdims=True)
        probs_ref[...] = probs
        y = jnp.log(probs + 1e-20) + g_ref[...]
        ymax = jnp.max(y, axis=1, keepdims=True)
        aio = lax.broadcasted_iota(jnp.int32, (B, A), 1)
        act_ref[...] = jnp.min(jnp.where(y == ymax, aio, A), axis=1)[None, :]

    return pl.pallas_call(
        body,
        out_shape=(
            jax.ShapeDtypeStruct((B, A), jnp.float32),
            jax.ShapeDtypeStruct((1, B), jnp.int32),
        ),
    )(out_state, w_out_t, b_out.reshape(1, A), gumbel)


def kernel(utterance, global_idxes, emb_table, W_ih, W_hh, b_ih, b_hh, W_out, b_out):
    del global_idxes
    tokens_tm = utterance.T.reshape(-1)
    wih = W_ih.T.astype(jnp.bfloat16)
    whh = W_hh.T.astype(jnp.bfloat16)
    e = _first_zero(utterance)
    h = jnp.zeros((B, D), jnp.float32)
    out = jnp.zeros((B, D), jnp.float32)
    base = 0
    for ct in (64, 128, 128, 192):
        emb_c = _sc_gather(emb_table, tokens_tm[base * B:(base + ct) * B])
        h, out = _chunk_fwd(base, ct, e, emb_c, wih, b_ih, whh, b_hh, h, out)
        base += ct
    gumbel = jax.random.gumbel(jax.random.key(42), (B, A), jnp.float32)
    probs, act = _finalize(out, W_out.T.astype(jnp.bfloat16), b_out, gumbel)
    return probs, act.reshape(B)


# speedup vs baseline: 1.0097x; 1.0031x over previous
import functools

import jax
import jax.numpy as jnp
from jax import lax
from jax.experimental import pallas as pl
from jax.experimental.pallas import tpu as pltpu
from jax.experimental.pallas import tpu_sc as plsc

B = 16
T = 512
D = 512
H3 = 3 * D
A = 256

_NC = 2
_NS = 16
_NW = _NC * _NS

_PREC = lax.Precision.DEFAULT


def _sc_gather(table, idx):
    n = idx.shape[0]
    d = table.shape[1]
    per_w = n // _NW
    chunk = max(c for c in range(1, min(per_w, 64) + 1) if per_w % c == 0)
    mesh = plsc.VectorSubcoreMesh(core_axis_name="c", subcore_axis_name="s")

    @functools.partial(
        pl.kernel,
        mesh=mesh,
        out_type=jax.ShapeDtypeStruct((n, d), table.dtype),
        scratch_types=[
            pltpu.VMEM((per_w,), jnp.int32),
            pltpu.VMEM((chunk, d), table.dtype),
            pltpu.VMEM((chunk, d), table.dtype),
            pltpu.SemaphoreType.DMA,
            pltpu.SemaphoreType.DMA,
        ],
    )
    def k(table_hbm, idx_hbm, out_hbm, idx_v, rows0, rows1, sem0, sem1):
        wid = lax.axis_index("s") * _NC + lax.axis_index("c")
        base = wid * per_w
        pltpu.sync_copy(idx_hbm.at[pl.ds(base, per_w)], idx_v)
        bufs = (rows0, rows1)
        sems = (sem0, sem1)
        nchunks = per_w // chunk
        pltpu.async_copy(table_hbm.at[idx_v.at[pl.ds(0, chunk)]], bufs[0], sems[0])
        for c in range(nchunks):
            if c + 1 < nchunks:
                pltpu.async_copy(
                    table_hbm.at[idx_v.at[pl.ds((c + 1) * chunk, chunk)]],
                    bufs[(c + 1) % 2], sems[(c + 1) % 2])
            pltpu.make_async_copy(
                table_hbm.at[idx_v.at[pl.ds(c * chunk, chunk)]],
                bufs[c % 2], sems[c % 2]).wait()
            pltpu.sync_copy(bufs[c % 2], out_hbm.at[pl.ds(base + c * chunk, chunk)])

    return k(table, idx)


def _sig(x):
    return 0.5 * jnp.tanh(0.5 * x) + 0.5


def _chunk_fwd(base0, ct, e_sm, emb, w_ih_t, b_ih, w_hh_t, b_hh, h0, out0):
    st = 64
    mm_rows = st * B
    ngrid = ct // st

    def body(e_ref, emb_ref, wih_ref, bih_ref, whh_ref, bhh_ref,
             h0_ref, out0_ref, h1_ref, out1_ref, gi_ref, hall_ref, h_ref):
        g = pl.program_id(0)

        @pl.when(g == 0)
        def _():
            h_ref[...] = h0_ref[...]
            out1_ref[...] = out0_ref[...]

        gi = lax.dot_general(
            emb_ref[...].astype(jnp.bfloat16),
            wih_ref[...], (((1,), (0,)), ((), ())),
            preferred_element_type=jnp.float32, precision=_PREC,
        ) + bih_ref[...]
        gi_ref[...] = gi.astype(jnp.bfloat16).reshape(st, B, H3)

        def step(i, h):
            gg = gi_ref[i].astype(jnp.float32)
            hb = h.astype(jnp.bfloat16)
            gh_rz = lax.dot_general(
                hb, whh_ref[:, :2 * D], (((1,), (0,)), ((), ())),
                preferred_element_type=jnp.float32, precision=_PREC,
            ) + bhh_ref[:, :2 * D]
            gh_n = lax.dot_general(
                hb, whh_ref[:, 2 * D:], (((1,), (0,)), ((), ())),
                preferred_element_type=jnp.float32, precision=_PREC,
            ) + bhh_ref[:, 2 * D:]
            r = _sig(gg[:, :D] + gh_rz[:, :D])
            z = _sig(gg[:, D:2 * D] + gh_rz[:, D:2 * D])
            nn = jnp.tanh(gg[:, 2 * D:] + r * gh_n)
            h = (1.0 - z) * nn + z * h
            hall_ref[i, :, :] = h
            return h

        h = lax.fori_loop(0, st, step, h_ref[...], unroll=4)
        h_ref[...] = h
        h1_ref[...] = h
        base = base0 + g * st
        for b in range(B):
            eb = e_ref[0, b]

            @pl.when(jnp.logical_and(eb >= base, eb < base + st))
            def _():
                out1_ref[b, :] = hall_ref[eb - base, b, :]

    return pl.pallas_call(
        body,
        grid=(ngrid,),
        in_specs=[
            pl.BlockSpec(memory_space=pltpu.SMEM),
            pl.BlockSpec((mm_rows, D), lambda g: (g, 0)),
            pl.BlockSpec((D, H3), lambda g: (0, 0)),
            pl.BlockSpec((1, H3), lambda g: (0, 0)),
            pl.BlockSpec((D, H3), lambda g: (0, 0)),
            pl.BlockSpec((1, H3), lambda g: (0, 0)),
            pl.BlockSpec((B, D), lambda g: (0, 0)),
            pl.BlockSpec((B, D), lambda g: (0, 0)),
        ],
        out_specs=(
            pl.BlockSpec((B, D), lambda g: (0, 0)),
            pl.BlockSpec((B, D), lambda g: (0, 0)),
        ),
        out_shape=(
            jax.ShapeDtypeStruct((B, D), jnp.float32),
            jax.ShapeDtypeStruct((B, D), jnp.float32),
        ),
        scratch_shapes=[
            pltpu.VMEM((st, B, H3), jnp.bfloat16),
            pltpu.VMEM((st, B, D), jnp.float32),
            pltpu.VMEM((B, D), jnp.float32),
        ],
    )(e_sm, emb, w_ih_t, b_ih.reshape(1, H3), w_hh_t, b_hh.reshape(1, H3),
      h0, out0)


def _first_zero(utterance):
    def body(u_ref, e_ref):
        u = u_ref[...]
        iota = lax.broadcasted_iota(jnp.int32, (B, T), 1)
        cand = jnp.where(u == 0, iota, T - 1)
        e_ref[...] = jnp.min(cand, axis=1)[None, :]

    return pl.pallas_call(
        body, out_shape=jax.ShapeDtypeStruct((1, B), jnp.int32)
    )(utterance)


def _finalize(out_state, w_out_t, b_out, gumbel):
    def body(h_ref, w_ref, b_ref, g_ref, probs_ref, act_ref):
        logits = lax.dot_general(
            h_ref[...].astype(jnp.bfloat16), w_ref[...],
            (((1,), (0,)), ((), ())),
            preferred_element_type=jnp.float32, precision=_PREC,
        ) + b_ref[...]
        mx = jnp.max(logits, axis=1, keepdims=True)
        ex = jnp.exp(logits - mx)
        probs = ex / jnp.sum(ex, axis=1, keepdims=True)
        probs_ref[...] = probs
        y = jnp.log(probs + 1e-20) + g_ref[...]
        ymax = jnp.max(y, axis=1, keepdims=True)
        aio = lax.broadcasted_iota(jnp.int32, (B, A), 1)
        act_ref[...] = jnp.min(jnp.where(y == ymax, aio, A), axis=1)[None, :]

    return pl.pallas_call(
        body,
        out_shape=(
            jax.ShapeDtypeStruct((B, A), jnp.float32),
            jax.ShapeDtypeStruct((1, B), jnp.int32),
        ),
    )(out_state, w_out_t, b_out.reshape(1, A), gumbel)


def kernel(utterance, global_idxes, emb_table, W_ih, W_hh, b_ih, b_hh, W_out, b_out):
    del global_idxes
    tokens_tm = utterance.T.reshape(-1)
    wih = W_ih.T.astype(jnp.bfloat16)
    whh = W_hh.T.astype(jnp.bfloat16)
    e = _first_zero(utterance)
    h = jnp.zeros((B, D), jnp.float32)
    out = jnp.zeros((B, D), jnp.float32)
    base = 0
    for ct in (64, 128, 128, 192):
        emb_c = _sc_gather(emb_table, tokens_tm[base * B:(base + ct) * B])
        h, out = _chunk_fwd(base, ct, e, emb_c, wih, b_ih, whh, b_hh, h, out)
        base += ct
    gumbel = jax.random.gumbel(jax.random.key(42), (B, A), jnp.float32)
    probs, act = _finalize(out, W_out.T.astype(jnp.bfloat16), b_out, gumbel)
    return probs, act.reshape(B)
